# baseline (speedup 1.0000x reference)
import functools

import jax
import jax.numpy as jnp
from jax import lax
from jax.experimental import pallas as pl
from jax.experimental.pallas import tpu as pltpu
from jax.experimental.pallas import tpu_sc as plsc

T = 200
B = 4096
D = 64
VOCAB = 1000000
N = T * B
NC = 2
NS = 16
NW = NC * NS
NPW = N // NW
CHUNK = 512
NCHUNK = NPW // CHUNK
NSTEP = NCHUNK // 2

NFULL = VOCAB // 128
NPAIR = (NFULL // NW) // 2
NREM = NFULL - NW * 2 * NPAIR
TAILV = NFULL * 128

_mesh = plsc.VectorSubcoreMesh(core_axis_name="c", subcore_axis_name="s")


@functools.partial(
    pl.kernel,
    out_type=jax.ShapeDtypeStruct((VOCAB, 128), jnp.float32),
    mesh=_mesh,
    scratch_types=[
        pltpu.VMEM((D, 129), jnp.float32),
        pltpu.VMEM((D, 129), jnp.float32),
        pltpu.VMEM((128, 128), jnp.float32),
        pltpu.VMEM((128, 128), jnp.float32),
        pltpu.SemaphoreType.DMA,
        pltpu.SemaphoreType.DMA,
        pltpu.SemaphoreType.DMA,
        pltpu.SemaphoreType.DMA,
    ],
    compiler_params=pltpu.CompilerParams(
        use_tc_tiling_on_sc=True, needs_layout_passes=False),
)
def _transpose_kernel(tt_hbm, tail_hbm, out_hbm, s0, s1, t0, t1,
                      rsem0, rsem1, wsem0, wsem1):
    wid = lax.axis_index("s") * NC + lax.axis_index("c")
    iota = lax.iota(jnp.int32, 16)
    fvecs = [iota + 16 * q for q in range(D // 16)]

    def start_read(sbuf, sem, blk):
        pltpu.async_copy(tt_hbm.at[:, pl.ds(blk * 128, 128)],
                         sbuf.at[:, pl.ds(0, 128)], sem)

    def wait_read(sbuf, sem):
        pltpu.make_async_copy(tt_hbm.at[:, pl.ds(0, 128)],
                              sbuf.at[:, pl.ds(0, 128)], sem).wait()

    def transpose(sbuf, tbuf):
        def vbody(v0, carry):
            vals = []
            for dv in range(8):
                bv = iota * 0 + (8 * v0 + dv)
                for q in range(D // 16):
                    vals.append(plsc.load_gather(sbuf, [fvecs[q], bv]))
            k = 0
            for dv in range(8):
                v = 8 * v0 + dv
                for q in range(D // 16):
                    tbuf[v, pl.ds(16 * q, 16)] = vals[k]
                    k += 1
            return carry
        lax.fori_loop(0, 16, vbody, 0)

    def start_wb(tbuf, sem, blk):
        pltpu.async_copy(tbuf, out_hbm.at[pl.ds(blk * 128, 128), :], sem)

    def wait_wb(tbuf, sem):
        pltpu.make_async_copy(
            tbuf, out_hbm.at[pl.ds(0, 128), :], sem).wait()

    start_read(s0, rsem0, wid)
    start_read(s1, rsem1, wid + NW)

    def body(s, carry):
        b0 = wid + NW * 2 * s
        wait_read(s0, rsem0)
        transpose(s0, t0)
        start_read(s0, rsem0, lax.min(b0 + 2 * NW, NFULL - 1))
        wait_wb(t0, wsem0)
        start_wb(t0, wsem0, b0)
        wait_read(s1, rsem1)
        transpose(s1, t1)
        start_read(s1, rsem1, lax.min(b0 + 3 * NW, NFULL - 1))
        wait_wb(t1, wsem1)
        start_wb(t1, wsem1, b0 + NW)
        return carry

    wait_read(s0, rsem0)
    transpose(s0, t0)
    start_read(s0, rsem0, wid + 2 * NW)
    start_wb(t0, wsem0, wid)
    wait_read(s1, rsem1)
    transpose(s1, t1)
    start_read(s1, rsem1, wid + 3 * NW)
    start_wb(t1, wsem1, wid + NW)
    lax.fori_loop(1, NPAIR, body, 0)

    wait_read(s0, rsem0)
    wait_read(s1, rsem1)
    wait_wb(t0, wsem0)
    wait_wb(t1, wsem1)

    @pl.when(wid < NREM)
    def _():
        blk = NW * 2 * NPAIR + wid
        start_read(s0, rsem0, blk)
        wait_read(s0, rsem0)
        transpose(s0, t0)
        start_wb(t0, wsem0, blk)
        wait_wb(t0, wsem0)

    @pl.when(wid == NREM)
    def _():
        pltpu.async_copy(tail_hbm, s0.at[:, pl.ds(0, 128)], rsem0)
        pltpu.make_async_copy(tail_hbm, s0.at[:, pl.ds(0, 128)], rsem0).wait()
        pltpu.async_copy(s0.at[:, pl.ds(0, 128)],
                         out_hbm.at[pl.ds(TAILV, D), :], wsem0)
        pltpu.make_async_copy(s0.at[:, pl.ds(0, 128)],
                              out_hbm.at[pl.ds(TAILV, D), :], wsem0).wait()


@functools.partial(
    pl.kernel,
    out_type=jax.ShapeDtypeStruct((T, B, 2 * D), jnp.float32),
    mesh=_mesh,
    scratch_types=[
        pltpu.VMEM((CHUNK,), jnp.int32),
        pltpu.VMEM((CHUNK,), jnp.int32),
        pltpu.VMEM((CHUNK, D), jnp.float32),
        pltpu.VMEM((CHUNK, D), jnp.float32),
        pltpu.SemaphoreType.DMA,
        pltpu.SemaphoreType.DMA,
        pltpu.SemaphoreType.DMA,
        pltpu.SemaphoreType.DMA,
        pltpu.SemaphoreType.DMA,
        pltpu.SemaphoreType.DMA,
    ],
    compiler_params=pltpu.CompilerParams(
        use_tc_tiling_on_sc=False, needs_layout_passes=False),
)
def _gather_kernel(idx_hbm, table_hbm, out_hbm, idx0, idx1, rows0, rows1,
                   isem0, isem1, gsem0, gsem1, wsem0, wsem1):
    wid = lax.axis_index("s") * NC + lax.axis_index("c")
    base = wid * NPW

    def start_idx(buf, sem, chunk):
        off = base + lax.min(chunk, NCHUNK - 1) * CHUNK
        pltpu.async_copy(idx_hbm.at[pl.ds(off, CHUNK)], buf, sem)

    def wait_idx(buf, sem):
        pltpu.make_async_copy(idx_hbm.at[pl.ds(base, CHUNK)], buf, sem).wait()

    def start_gather(ibuf, rbuf, sem):
        return pltpu.async_copy(table_hbm.at[ibuf], rbuf, sem)

    def start_wb(rbuf, sem, chunk):
        off = base + chunk * CHUNK
        t = off // B
        b = off % B
        pltpu.async_copy(rbuf, out_hbm.at[t, pl.ds(b, CHUNK), pl.ds(0, D)],
                         sem)

    def wait_wb(rbuf, sem):
        pltpu.make_async_copy(
            rbuf, out_hbm.at[0, pl.ds(0, CHUNK), pl.ds(0, D)], sem).wait()

    start_idx(idx0, isem0, 0)
    start_idx(idx1, isem1, 1)
    wait_idx(idx0, isem0)
    g0 = start_gather(idx0, rows0, gsem0)
    wait_idx(idx1, isem1)
    g1 = start_gather(idx1, rows1, gsem1)
    g0.wait()
    start_wb(rows0, wsem0, 0)
    start_idx(idx0, isem0, 2)
    g1.wait()
    start_wb(rows1, wsem1, 1)
    start_idx(idx1, isem1, 3)

    def body(s, carry):
        c0 = 2 * s
        wait_idx(idx0, isem0)
        wait_wb(rows0, wsem0)
        d0 = start_gather(idx0, rows0, gsem0)
        wait_idx(idx1, isem1)
        wait_wb(rows1, wsem1)
        d1 = start_gather(idx1, rows1, gsem1)
        d0.wait()
        start_wb(rows0, wsem0, c0)
        start_idx(idx0, isem0, c0 + 2)
        d1.wait()
        start_wb(rows1, wsem1, c0 + 1)
        start_idx(idx1, isem1, c0 + 3)
        return carry

    lax.fori_loop(1, NSTEP, body, 0)

    wait_wb(rows0, wsem0)
    wait_wb(rows1, wsem1)
    wait_idx(idx0, isem0)
    wait_idx(idx1, isem1)


def kernel(x, table):
    tail128 = jnp.pad(table[TAILV:, :], ((0, 0), (0, D)))
    t128 = _transpose_kernel(table.T, tail128)
    t2 = t128.reshape(2 * VOCAB, D)
    flat2 = x.reshape(-1) * 2
    out128 = _gather_kernel(flat2, t2)
    return out128[:, :, :D]


# speedup vs baseline: 1.5694x; 1.5694x over previous
import functools

import jax
import jax.numpy as jnp
from jax import lax
from jax.experimental import pallas as pl
from jax.experimental.pallas import tpu as pltpu
from jax.experimental.pallas import tpu_sc as plsc

T = 200
B = 4096
D = 64
VOCAB = 1000000
N = T * B
NC = 2
NS = 16
NW = NC * NS
NPW = N // NW
CHUNK = 512
NCHUNK = NPW // CHUNK
NSTEP = NCHUNK // 2

_mesh = plsc.VectorSubcoreMesh(core_axis_name="c", subcore_axis_name="s")


@functools.partial(
    pl.kernel,
    out_type=jax.ShapeDtypeStruct((T, B, 2 * D), jnp.float32),
    mesh=_mesh,
    scratch_types=[
        pltpu.VMEM((CHUNK,), jnp.int32),
        pltpu.VMEM((CHUNK,), jnp.int32),
        pltpu.VMEM((CHUNK, D), jnp.float32),
        pltpu.VMEM((CHUNK, D), jnp.float32),
        pltpu.SemaphoreType.DMA,
        pltpu.SemaphoreType.DMA,
        pltpu.SemaphoreType.DMA,
        pltpu.SemaphoreType.DMA,
        pltpu.SemaphoreType.DMA,
        pltpu.SemaphoreType.DMA,
    ],
    compiler_params=pltpu.CompilerParams(
        use_tc_tiling_on_sc=False, needs_layout_passes=False),
)
def _gather_kernel(idx_hbm, table_hbm, out_hbm, idx0, idx1, rows0, rows1,
                   isem0, isem1, gsem0, gsem1, wsem0, wsem1):
    wid = lax.axis_index("s") * NC + lax.axis_index("c")
    base = wid * NPW

    def start_idx(buf, sem, chunk):
        off = base + lax.min(chunk, NCHUNK - 1) * CHUNK
        pltpu.async_copy(idx_hbm.at[pl.ds(off, CHUNK)], buf, sem)

    def wait_idx(buf, sem):
        pltpu.make_async_copy(idx_hbm.at[pl.ds(base, CHUNK)], buf, sem).wait()

    def start_gather(ibuf, rbuf, sem):
        return pltpu.async_copy(table_hbm.at[ibuf], rbuf, sem)

    def start_wb(rbuf, sem, chunk):
        off = base + chunk * CHUNK
        t = off // B
        b = off % B
        pltpu.async_copy(rbuf, out_hbm.at[t, pl.ds(b, CHUNK), pl.ds(0, D)],
                         sem)

    def wait_wb(rbuf, sem):
        pltpu.make_async_copy(
            rbuf, out_hbm.at[0, pl.ds(0, CHUNK), pl.ds(0, D)], sem).wait()

    start_idx(idx0, isem0, 0)
    start_idx(idx1, isem1, 1)
    wait_idx(idx0, isem0)
    g0 = start_gather(idx0, rows0, gsem0)
    wait_idx(idx1, isem1)
    g1 = start_gather(idx1, rows1, gsem1)
    g0.wait()
    start_wb(rows0, wsem0, 0)
    start_idx(idx0, isem0, 2)
    g1.wait()
    start_wb(rows1, wsem1, 1)
    start_idx(idx1, isem1, 3)

    def body(s, carry):
        c0 = 2 * s
        wait_idx(idx0, isem0)
        wait_wb(rows0, wsem0)
        d0 = start_gather(idx0, rows0, gsem0)
        wait_idx(idx1, isem1)
        wait_wb(rows1, wsem1)
        d1 = start_gather(idx1, rows1, gsem1)
        d0.wait()
        start_wb(rows0, wsem0, c0)
        start_idx(idx0, isem0, c0 + 2)
        d1.wait()
        start_wb(rows1, wsem1, c0 + 1)
        start_idx(idx1, isem1, c0 + 3)
        return carry

    lax.fori_loop(1, NSTEP, body, 0)

    wait_wb(rows0, wsem0)
    wait_wb(rows1, wsem1)
    wait_idx(idx0, isem0)
    wait_idx(idx1, isem1)


def kernel(x, table):
    flat = x.reshape(-1)
    out128 = _gather_kernel(flat, table)
    return out128[:, :, :D]
